# Initial kernel scaffold; baseline (speedup 1.0000x reference)
import jax
import jax.numpy as jnp
from jax.experimental import pallas as pl


def kernel(x, edge_index, enc_W1, enc_b1, enc_W2, enc_b2, enc_g, enc_beta, proc_Wl1, proc_bl1, proc_Wr1, proc_Wl2, proc_bl2, proc_Wr2, proc_g, proc_beta, dec_W1, dec_b1, dec_W2, dec_b2):
    raise NotImplementedError("write your pallas kernel here")


# speedup vs baseline: 1.0000x; 1.0000x over previous
import jax
import jax.numpy as jnp
from jax import lax
from jax.experimental import pallas as pl


def _identity_kernel(x_ref, o_ref):
    o_ref[...] = x_ref[...]


def _layer_norm(h, g, b, eps=1e-5):
    mu = jnp.mean(h, axis=-1, keepdims=True)
    var = jnp.var(h, axis=-1, keepdims=True)
    return (h - mu) / jnp.sqrt(var + eps) * g + b


def _sage_conv(h, src, dst, Wl, bl, Wr, n_nodes):
    msg = jnp.take(h, src, axis=0)
    agg = jax.ops.segment_max(msg, dst, num_segments=n_nodes)
    agg = jnp.where(jnp.isinf(agg), 0.0, agg)
    return agg @ Wl + bl + h @ Wr


def kernel(x, edge_index, enc_W1, enc_b1, enc_W2, enc_b2, enc_g, enc_beta, proc_Wl1, proc_bl1, proc_Wr1, proc_Wl2, proc_bl2, proc_Wr2, proc_g, proc_beta, dec_W1, dec_b1, dec_W2, dec_b2):
    n_nodes = x.shape[0]
    x = pl.pallas_call(
        _identity_kernel,
        out_shape=jax.ShapeDtypeStruct(x.shape, x.dtype),
    )(x)
    src = edge_index[0]
    dst = edge_index[1]
    h = jax.nn.relu(x @ enc_W1 + enc_b1)
    h = h @ enc_W2 + enc_b2
    h = _layer_norm(h, enc_g, enc_beta)
    for i in range(3):
        h1 = jax.nn.relu(_sage_conv(h, src, dst, proc_Wl1[i], proc_bl1[i], proc_Wr1[i], n_nodes))
        h2 = _sage_conv(h1, src, dst, proc_Wl2[i], proc_bl2[i], proc_Wr2[i], n_nodes)
        h2 = h2 + h
        h = _layer_norm(h2, proc_g[i], proc_beta[i])
    d = jax.nn.relu(h @ dec_W1 + dec_b1)
    return d @ dec_W2 + dec_b2


# speedup vs baseline: 2.2564x; 2.2563x over previous
import functools

import jax
import jax.numpy as jnp
from jax import lax
from jax.experimental import pallas as pl
from jax.experimental.pallas import tpu as pltpu
from jax.experimental.pallas import tpu_sc as plsc

N = 10000
E = 320000
F = 128
NW = 32
ROWS = 320
NPAD = NW * ROWS
K = 64
BLK = 2000


def _build_seg_max():
    mesh = plsc.VectorSubcoreMesh(core_axis_name="c", subcore_axis_name="s")

    @functools.partial(
        pl.kernel,
        mesh=mesh,
        compiler_params=pltpu.CompilerParams(needs_layout_passes=False),
        out_type=jax.ShapeDtypeStruct((NPAD, F), jnp.float32),
        scratch_types=[
            pltpu.VMEM((ROWS, F), jnp.float32),
            pltpu.VMEM((K, F), jnp.float32),
            pltpu.VMEM((K,), jnp.int32),
            pltpu.VMEM((K,), jnp.int32),
            pltpu.VMEM((32,), jnp.int32),
            pltpu.SemaphoreType.DMA,
        ],
    )
    def seg_max(h_hbm, packed_hbm, se_hbm, out_hbm, agg, msg, pkd, gidx, se, sem):
        cid = lax.axis_index("c")
        sid = lax.axis_index("s")
        t = sid * 2 + cid

        pltpu.sync_copy(se_hbm.at[pl.ds(pl.multiple_of(t * 32, 32), 32)], se)
        sev0 = se[0:16]
        sev1 = se[16:32]
        start = sev0[0]
        end = sev1[0]
        start8 = pl.multiple_of(start & jnp.int32(-8), 8)
        nch = (end - start8 + (K - 1)) >> 6

        zv = jnp.zeros((16,), jnp.float32)

        def zrow(i, carry):
            for f in range(8):
                agg[i, pl.ds(f * 16, 16)] = zv
            return carry

        lax.fori_loop(0, ROWS, zrow, 0)

        negv = jnp.full((16,), -jnp.inf, jnp.float32)
        iota = lax.iota(jnp.int32, 16)
        startv = jnp.full((16,), start, jnp.int32)
        endv = jnp.full((16,), end, jnp.int32)

        def chunk_body(c, acc):
            off = pl.multiple_of(start8 + c * K, 8)
            pltpu.sync_copy(packed_hbm.at[pl.ds(off, K)], pkd)
            for q in range(K // 16):
                gidx[pl.ds(q * 16, 16)] = pkd[pl.ds(q * 16, 16)] >> 10
            pltpu.async_copy(h_hbm.at[gidx], msg, sem).wait()

            def edge_body(j, acc):
                jv = jnp.full((16,), j, jnp.int32)
                m = plsc.load_gather(pkd, [jv])
                ev = jnp.full((16,), off + j, jnp.int32)
                vmask = (ev >= startv) & (ev < endv)
                firstm = (m & 1) > 0
                row = (m >> 1) & 511
                nacc = []
                for f in range(8):
                    v = msg[j, pl.ds(f * 16, 16)]
                    a = jnp.maximum(jnp.where(firstm, negv, acc[f]), v)
                    plsc.store_scatter(agg, [row, iota + f * 16], a, mask=vmask)
                    nacc.append(a)
                return tuple(nacc)

            return lax.fori_loop(0, K, edge_body, acc)

        lax.fori_loop(0, nch, chunk_body, (negv,) * 8)
        pltpu.sync_copy(agg, out_hbm.at[pl.ds(pl.multiple_of(t * ROWS, ROWS), ROWS)])

    return seg_max


_seg_max = _build_seg_max()


def _ln(h, g, b):
    mu = jnp.mean(h, axis=-1, keepdims=True)
    d = h - mu
    var = jnp.mean(d * d, axis=-1, keepdims=True)
    return d * lax.rsqrt(var + 1e-5) * g + b


def _mlp_body(x_ref, w1_ref, b1_ref, w2_ref, b2_ref, g_ref, beta_ref, o_ref, *, ln):
    h = jnp.dot(x_ref[...], w1_ref[...], preferred_element_type=jnp.float32)
    h = jnp.maximum(h + b1_ref[...], 0.0)
    h = jnp.dot(h, w2_ref[...], preferred_element_type=jnp.float32) + b2_ref[...]
    if ln:
        h = _ln(h, g_ref[...], beta_ref[...])
    o_ref[...] = h


def _mlp(x, w1, b1, w2, b2, g, beta, ln):
    full = lambda i: (0, 0)
    return pl.pallas_call(
        functools.partial(_mlp_body, ln=ln),
        grid=(N // BLK,),
        in_specs=[
            pl.BlockSpec((BLK, F), lambda i: (i, 0)),
            pl.BlockSpec((F, F), full),
            pl.BlockSpec((1, F), full),
            pl.BlockSpec((F, F), full),
            pl.BlockSpec((1, F), full),
            pl.BlockSpec((1, F), full),
            pl.BlockSpec((1, F), full),
        ],
        out_specs=pl.BlockSpec((BLK, F), lambda i: (i, 0)),
        out_shape=jax.ShapeDtypeStruct((N, F), jnp.float32),
    )(x, w1, b1.reshape(1, F), w2, b2.reshape(1, F), g.reshape(1, F), beta.reshape(1, F))


def _combine1_body(agg_ref, h_ref, wl_ref, wr_ref, b_ref, o_ref):
    v = jnp.dot(agg_ref[...], wl_ref[...], preferred_element_type=jnp.float32)
    v += jnp.dot(h_ref[...], wr_ref[...], preferred_element_type=jnp.float32)
    o_ref[...] = jnp.maximum(v + b_ref[...], 0.0)


def _combine1(agg_pad, h, wl, wr, b):
    full = lambda i: (0, 0)
    return pl.pallas_call(
        _combine1_body,
        grid=(N // BLK,),
        in_specs=[
            pl.BlockSpec((BLK, F), lambda i: (i, 0)),
            pl.BlockSpec((BLK, F), lambda i: (i, 0)),
            pl.BlockSpec((F, F), full),
            pl.BlockSpec((F, F), full),
            pl.BlockSpec((1, F), full),
        ],
        out_specs=pl.BlockSpec((BLK, F), lambda i: (i, 0)),
        out_shape=jax.ShapeDtypeStruct((N, F), jnp.float32),
    )(agg_pad, h, wl, wr, b.reshape(1, F))


def _combine2_body(agg_ref, h1_ref, res_ref, wl_ref, wr_ref, b_ref, g_ref, beta_ref, o_ref):
    v = jnp.dot(agg_ref[...], wl_ref[...], preferred_element_type=jnp.float32)
    v += jnp.dot(h1_ref[...], wr_ref[...], preferred_element_type=jnp.float32)
    v += b_ref[...] + res_ref[...]
    o_ref[...] = _ln(v, g_ref[...], beta_ref[...])


def _combine2(agg_pad, h1, res, wl, wr, b, g, beta):
    full = lambda i: (0, 0)
    return pl.pallas_call(
        _combine2_body,
        grid=(N // BLK,),
        in_specs=[
            pl.BlockSpec((BLK, F), lambda i: (i, 0)),
            pl.BlockSpec((BLK, F), lambda i: (i, 0)),
            pl.BlockSpec((BLK, F), lambda i: (i, 0)),
            pl.BlockSpec((F, F), full),
            pl.BlockSpec((F, F), full),
            pl.BlockSpec((1, F), full),
            pl.BlockSpec((1, F), full),
            pl.BlockSpec((1, F), full),
        ],
        out_specs=pl.BlockSpec((BLK, F), lambda i: (i, 0)),
        out_shape=jax.ShapeDtypeStruct((N, F), jnp.float32),
    )(agg_pad, h1, res, wl, wr, b.reshape(1, F), g.reshape(1, F), beta.reshape(1, F))


def kernel(x, edge_index, enc_W1, enc_b1, enc_W2, enc_b2, enc_g, enc_beta,
           proc_Wl1, proc_bl1, proc_Wr1, proc_Wl2, proc_bl2, proc_Wr2,
           proc_g, proc_beta, dec_W1, dec_b1, dec_W2, dec_b2):
    src = edge_index[0]
    dst = edge_index[1]

    dst_s, src_s = lax.sort((dst, src), num_keys=1)
    first = jnp.concatenate([
        jnp.ones((1,), jnp.int32),
        (dst_s[1:] != dst_s[:-1]).astype(jnp.int32),
    ])
    row = jnp.mod(dst_s, ROWS)
    packed = (src_s << 10) | (row << 1) | first
    packed = jnp.concatenate([packed, jnp.zeros((K + 8,), jnp.int32)])

    bounds = (ROWS * jnp.arange(33, dtype=jnp.int32)).astype(dst_s.dtype)
    edges_at = jnp.searchsorted(dst_s, bounds, side="left").astype(jnp.int32)
    starts, ends = edges_at[:32], edges_at[1:33]
    se_arr = jnp.concatenate(
        [jnp.repeat(starts[:, None], 16, 1), jnp.repeat(ends[:, None], 16, 1)],
        axis=1,
    ).reshape(-1)

    ones = jnp.ones((F,), jnp.float32)
    zeros = jnp.zeros((F,), jnp.float32)

    h = _mlp(x, enc_W1, enc_b1, enc_W2, enc_b2, enc_g, enc_beta, ln=True)
    for i in range(3):
        agg1 = _seg_max(h, packed, se_arr)
        h1 = _combine1(agg1, h, proc_Wl1[i], proc_Wr1[i], proc_bl1[i])
        agg2 = _seg_max(h1, packed, se_arr)
        h = _combine2(agg2, h1, h, proc_Wl2[i], proc_Wr2[i], proc_bl2[i],
                      proc_g[i], proc_beta[i])
    return _mlp(h, dec_W1, dec_b1, dec_W2, dec_b2, ones, zeros, ln=False)


# speedup vs baseline: 3.2357x; 1.4340x over previous
import functools

import jax
import jax.numpy as jnp
from jax import lax
from jax.experimental import pallas as pl
from jax.experimental.pallas import tpu as pltpu
from jax.experimental.pallas import tpu_sc as plsc

N = 10000
E = 320000
F = 128
NW = 32
ROWS = 320
NPAD = NW * ROWS
CAP = 16384
CH = 128
BLK = 2000


def _build_seg_max():
    mesh = plsc.VectorSubcoreMesh(core_axis_name="c", subcore_axis_name="s")

    @functools.partial(
        pl.kernel,
        mesh=mesh,
        compiler_params=pltpu.CompilerParams(needs_layout_passes=False),
        out_type=jax.ShapeDtypeStruct((NPAD, F), jnp.float32),
        scratch_types=[
            pltpu.VMEM((ROWS, F), jnp.float32),
            pltpu.VMEM((CH, F), jnp.float32),
            pltpu.VMEM((CH, F), jnp.float32),
            pltpu.VMEM((CAP,), jnp.int32),
            pltpu.VMEM((CAP,), jnp.int32),
            pltpu.VMEM((32,), jnp.int32),
            pltpu.SemaphoreType.DMA,
            pltpu.SemaphoreType.DMA,
        ],
    )
    def seg_max(h_hbm, src_hbm, meta_hbm, se_hbm, out_hbm,
                agg, msgA, msgB, srcs, meta, se, semA, semB):
        cid = lax.axis_index("c")
        sid = lax.axis_index("s")
        t = sid * 2 + cid

        pltpu.sync_copy(se_hbm.at[pl.ds(pl.multiple_of(t * 32, 32), 32)], se)
        sev0 = se[0:16]
        sev1 = se[16:32]
        start = sev0[0]
        end = sev1[0]
        start8 = pl.multiple_of(start & jnp.int32(-8), 8)
        nsc = (end - start8 + (CAP - 1)) >> 14

        zv = jnp.zeros((16,), jnp.float32)

        def zrow(i, carry):
            for f in range(8):
                agg[i, pl.ds(f * 16, 16)] = zv
            return carry

        lax.fori_loop(0, ROWS, zrow, 0)

        negv = jnp.full((16,), -jnp.inf, jnp.float32)
        iota = lax.iota(jnp.int32, 16)
        startv = jnp.full((16,), start, jnp.int32)
        endv = jnp.full((16,), end, jnp.int32)

        def fire(c, msgbuf, sem):
            pltpu.async_copy(h_hbm.at[srcs.at[pl.ds(c * CH, CH)]], msgbuf, sem)

        def wait(c, msgbuf, sem):
            pltpu.make_async_copy(h_hbm.at[srcs.at[pl.ds(c * CH, CH)]], msgbuf, sem).wait()

        def sc_body(s, acc):
            soff = pl.multiple_of(start8 + s * CAP, 8)
            pltpu.sync_copy(src_hbm.at[pl.ds(soff, CAP)], srcs)
            pltpu.sync_copy(meta_hbm.at[pl.ds(soff, CAP)], meta)
            nch = jnp.minimum((end - soff + (CH - 1)) >> 7, CAP // CH)

            @pl.when(nch > 0)
            def _():
                fire(0, msgA, semA)

            def compute(c, msgbuf, acc):
                base = c * CH

                def e4(jj, acc):
                    nacc = acc
                    for u in range(4):
                        jl = jj * 4 + u
                        le = base + jl
                        jv = jnp.full((16,), le, jnp.int32)
                        m = plsc.load_gather(meta, [jv])
                        ev = jnp.full((16,), soff + le, jnp.int32)
                        vmask = (ev >= startv) & (ev < endv)
                        firstm = (m & 1) > 0
                        row = m >> 1
                        nacc2 = []
                        for f in range(8):
                            v = msgbuf[jl, pl.ds(f * 16, 16)]
                            a = jnp.maximum(jnp.where(firstm, negv, nacc[f]), v)
                            plsc.store_scatter(agg, [row, iota + f * 16], a, mask=vmask)
                            nacc2.append(a)
                        nacc = tuple(nacc2)
                    return nacc

                return lax.fori_loop(0, CH // 4, e4, acc)

            def pair(c2, acc):
                c0 = c2 * 2
                c1 = c0 + 1

                @pl.when(c1 < nch)
                def _():
                    fire(c1, msgB, semB)

                wait(c0, msgA, semA)
                acc = compute(c0, msgA, acc)

                @pl.when(c0 + 2 < nch)
                def _():
                    fire(c0 + 2, msgA, semA)

                def doB(a):
                    wait(c1, msgB, semB)
                    return compute(c1, msgB, a)

                return lax.cond(c1 < nch, doB, lambda a: a, acc)

            return lax.fori_loop(0, (nch + 1) >> 1, pair, acc)

        lax.fori_loop(0, nsc, sc_body, (negv,) * 8)
        pltpu.sync_copy(agg, out_hbm.at[pl.ds(pl.multiple_of(t * ROWS, ROWS), ROWS)])

    return seg_max


_seg_max = _build_seg_max()


def _ln(h, g, b):
    mu = jnp.mean(h, axis=-1, keepdims=True)
    d = h - mu
    var = jnp.mean(d * d, axis=-1, keepdims=True)
    return d * lax.rsqrt(var + 1e-5) * g + b


def _mlp_body(x_ref, w1_ref, b1_ref, w2_ref, b2_ref, g_ref, beta_ref, o_ref, *, ln):
    h = jnp.dot(x_ref[...], w1_ref[...], preferred_element_type=jnp.float32)
    h = jnp.maximum(h + b1_ref[...], 0.0)
    h = jnp.dot(h, w2_ref[...], preferred_element_type=jnp.float32) + b2_ref[...]
    if ln:
        h = _ln(h, g_ref[...], beta_ref[...])
    o_ref[...] = h


def _mlp(x, w1, b1, w2, b2, g, beta, ln):
    full = lambda i: (0, 0)
    return pl.pallas_call(
        functools.partial(_mlp_body, ln=ln),
        grid=(N // BLK,),
        in_specs=[
            pl.BlockSpec((BLK, F), lambda i: (i, 0)),
            pl.BlockSpec((F, F), full),
            pl.BlockSpec((1, F), full),
            pl.BlockSpec((F, F), full),
            pl.BlockSpec((1, F), full),
            pl.BlockSpec((1, F), full),
            pl.BlockSpec((1, F), full),
        ],
        out_specs=pl.BlockSpec((BLK, F), lambda i: (i, 0)),
        out_shape=jax.ShapeDtypeStruct((N, F), jnp.float32),
    )(x, w1, b1.reshape(1, F), w2, b2.reshape(1, F), g.reshape(1, F), beta.reshape(1, F))


def _combine1_body(agg_ref, h_ref, wl_ref, wr_ref, b_ref, o_ref):
    v = jnp.dot(agg_ref[...], wl_ref[...], preferred_element_type=jnp.float32)
    v += jnp.dot(h_ref[...], wr_ref[...], preferred_element_type=jnp.float32)
    o_ref[...] = jnp.maximum(v + b_ref[...], 0.0)


def _combine1(agg_pad, h, wl, wr, b):
    full = lambda i: (0, 0)
    return pl.pallas_call(
        _combine1_body,
        grid=(N // BLK,),
        in_specs=[
            pl.BlockSpec((BLK, F), lambda i: (i, 0)),
            pl.BlockSpec((BLK, F), lambda i: (i, 0)),
            pl.BlockSpec((F, F), full),
            pl.BlockSpec((F, F), full),
            pl.BlockSpec((1, F), full),
        ],
        out_specs=pl.BlockSpec((BLK, F), lambda i: (i, 0)),
        out_shape=jax.ShapeDtypeStruct((N, F), jnp.float32),
    )(agg_pad, h, wl, wr, b.reshape(1, F))


def _combine2_body(agg_ref, h1_ref, res_ref, wl_ref, wr_ref, b_ref, g_ref, beta_ref, o_ref):
    v = jnp.dot(agg_ref[...], wl_ref[...], preferred_element_type=jnp.float32)
    v += jnp.dot(h1_ref[...], wr_ref[...], preferred_element_type=jnp.float32)
    v += b_ref[...] + res_ref[...]
    o_ref[...] = _ln(v, g_ref[...], beta_ref[...])


def _combine2(agg_pad, h1, res, wl, wr, b, g, beta):
    full = lambda i: (0, 0)
    return pl.pallas_call(
        _combine2_body,
        grid=(N // BLK,),
        in_specs=[
            pl.BlockSpec((BLK, F), lambda i: (i, 0)),
            pl.BlockSpec((BLK, F), lambda i: (i, 0)),
            pl.BlockSpec((BLK, F), lambda i: (i, 0)),
            pl.BlockSpec((F, F), full),
            pl.BlockSpec((F, F), full),
            pl.BlockSpec((1, F), full),
            pl.BlockSpec((1, F), full),
            pl.BlockSpec((1, F), full),
        ],
        out_specs=pl.BlockSpec((BLK, F), lambda i: (i, 0)),
        out_shape=jax.ShapeDtypeStruct((N, F), jnp.float32),
    )(agg_pad, h1, res, wl, wr, b.reshape(1, F), g.reshape(1, F), beta.reshape(1, F))


def kernel(x, edge_index, enc_W1, enc_b1, enc_W2, enc_b2, enc_g, enc_beta,
           proc_Wl1, proc_bl1, proc_Wr1, proc_Wl2, proc_bl2, proc_Wr2,
           proc_g, proc_beta, dec_W1, dec_b1, dec_W2, dec_b2):
    src = edge_index[0]
    dst = edge_index[1]

    dst_s, src_s = lax.sort((dst, src), num_keys=1)
    first = jnp.concatenate([
        jnp.ones((1,), jnp.int32),
        (dst_s[1:] != dst_s[:-1]).astype(jnp.int32),
    ])
    row = jnp.mod(dst_s, ROWS)
    meta = (row << 1) | first
    pad = jnp.zeros((CAP,), jnp.int32)
    src_pad = jnp.concatenate([src_s, pad])
    meta_pad = jnp.concatenate([meta, pad])

    bounds = (ROWS * jnp.arange(33, dtype=jnp.int32)).astype(dst_s.dtype)
    edges_at = jnp.searchsorted(dst_s, bounds, side="left").astype(jnp.int32)
    starts, ends = edges_at[:32], edges_at[1:33]
    se_arr = jnp.concatenate(
        [jnp.repeat(starts[:, None], 16, 1), jnp.repeat(ends[:, None], 16, 1)],
        axis=1,
    ).reshape(-1)

    ones = jnp.ones((F,), jnp.float32)
    zeros = jnp.zeros((F,), jnp.float32)

    h = _mlp(x, enc_W1, enc_b1, enc_W2, enc_b2, enc_g, enc_beta, ln=True)
    for i in range(3):
        agg1 = _seg_max(h, src_pad, meta_pad, se_arr)
        h1 = _combine1(agg1, h, proc_Wl1[i], proc_Wr1[i], proc_bl1[i])
        agg2 = _seg_max(h1, src_pad, meta_pad, se_arr)
        h = _combine2(agg2, h1, h, proc_Wl2[i], proc_Wr2[i], proc_bl2[i],
                      proc_g[i], proc_beta[i])
    return _mlp(h, dec_W1, dec_b1, dec_W2, dec_b2, ones, zeros, ln=False)


# speedup vs baseline: 8.7235x; 2.6960x over previous
import functools

import jax
import jax.numpy as jnp
from jax import lax
from jax.experimental import pallas as pl
from jax.experimental.pallas import tpu as pltpu
from jax.experimental.pallas import tpu_sc as plsc

N = 10000
E = 320000
F = 128
NW = 32
ROWS = 320
NPAD = NW * ROWS
CAP = 16384
CH = 128
BLK = 2000


def _build_seg_max():
    mesh = plsc.VectorSubcoreMesh(core_axis_name="c", subcore_axis_name="s")

    @functools.partial(
        pl.kernel,
        mesh=mesh,
        compiler_params=pltpu.CompilerParams(needs_layout_passes=False),
        out_type=jax.ShapeDtypeStruct((NPAD, F), jnp.float32),
        scratch_types=[
            pltpu.VMEM((ROWS, F), jnp.float32),
            pltpu.VMEM((CH, F), jnp.float32),
            pltpu.VMEM((CH, F), jnp.float32),
            pltpu.VMEM((CAP,), jnp.int32),
            pltpu.VMEM((CAP,), jnp.int32),
            pltpu.VMEM((32,), jnp.int32),
            pltpu.SemaphoreType.DMA,
            pltpu.SemaphoreType.DMA,
        ],
    )
    def seg_max(h_hbm, src_hbm, meta_hbm, se_hbm, out_hbm,
                agg, msgA, msgB, srcs, meta, se, semA, semB):
        cid = lax.axis_index("c")
        sid = lax.axis_index("s")
        t = sid * 2 + cid

        pltpu.sync_copy(se_hbm.at[pl.ds(pl.multiple_of(t * 32, 32), 32)], se)
        sev0 = se[0:16]
        sev1 = se[16:32]
        start = sev0[0]
        end = sev1[0]
        start8 = pl.multiple_of(start & jnp.int32(-8), 8)
        nsc = (end - start8 + (CAP - 1)) >> 14

        zv = jnp.zeros((16,), jnp.float32)

        def zrow(i, carry):
            for f in range(8):
                agg[i, pl.ds(f * 16, 16)] = zv
            return carry

        lax.fori_loop(0, ROWS, zrow, 0)

        negv = jnp.full((16,), -jnp.inf, jnp.float32)
        iota = lax.iota(jnp.int32, 16)
        startv = jnp.full((16,), start, jnp.int32)
        endv = jnp.full((16,), end, jnp.int32)

        def fire(c, msgbuf, sem):
            pltpu.async_copy(h_hbm.at[srcs.at[pl.ds(c * CH, CH)]], msgbuf, sem)

        def wait(c, msgbuf, sem):
            pltpu.make_async_copy(h_hbm.at[srcs.at[pl.ds(c * CH, CH)]], msgbuf, sem).wait()

        def sc_body(s, acc):
            soff = pl.multiple_of(start8 + s * CAP, 8)
            pltpu.sync_copy(src_hbm.at[pl.ds(soff, CAP)], srcs)
            pltpu.sync_copy(meta_hbm.at[pl.ds(soff, CAP)], meta)
            nch = jnp.minimum((end - soff + (CH - 1)) >> 7, CAP // CH)

            @pl.when(nch > 0)
            def _():
                fire(0, msgA, semA)

            def compute(c, msgbuf, acc):
                base = c * CH

                def body(jl, acc):
                    le = base + jl
                    jv = jnp.full((16,), le, jnp.int32)
                    m = plsc.load_gather(meta, [jv])
                    ev = jnp.full((16,), soff + le, jnp.int32)
                    validm = (ev >= startv) & (ev < endv)
                    firstm = (m & 1) > 0
                    lastm = (m & 2) > 0
                    vmask = lastm & validm
                    row = m >> 2
                    nacc = []
                    for f in range(8):
                        v = msgbuf[jl, pl.ds(f * 16, 16)]
                        a = jnp.maximum(jnp.where(firstm, negv, acc[f]), v)
                        plsc.store_scatter(agg, [row, iota + f * 16], a, mask=vmask)
                        nacc.append(a)
                    return tuple(nacc)

                return plsc.parallel_loop(0, CH, 1, unroll=4, carry=acc)(body)

            def pair(c2, acc):
                c0 = c2 * 2
                c1 = c0 + 1

                @pl.when(c1 < nch)
                def _():
                    fire(c1, msgB, semB)

                wait(c0, msgA, semA)
                acc = compute(c0, msgA, acc)

                @pl.when(c0 + 2 < nch)
                def _():
                    fire(c0 + 2, msgA, semA)

                def doB(a):
                    wait(c1, msgB, semB)
                    return compute(c1, msgB, a)

                return lax.cond(c1 < nch, doB, lambda a: a, acc)

            return lax.fori_loop(0, (nch + 1) >> 1, pair, acc)

        lax.fori_loop(0, nsc, sc_body, (negv,) * 8)
        pltpu.sync_copy(agg, out_hbm.at[pl.ds(pl.multiple_of(t * ROWS, ROWS), ROWS)])

    return seg_max


_seg_max = _build_seg_max()


def _ln(h, g, b):
    mu = jnp.mean(h, axis=-1, keepdims=True)
    d = h - mu
    var = jnp.mean(d * d, axis=-1, keepdims=True)
    return d * lax.rsqrt(var + 1e-5) * g + b


def _mlp_body(x_ref, w1_ref, b1_ref, w2_ref, b2_ref, g_ref, beta_ref, o_ref, *, ln):
    h = jnp.dot(x_ref[...], w1_ref[...], preferred_element_type=jnp.float32)
    h = jnp.maximum(h + b1_ref[...], 0.0)
    h = jnp.dot(h, w2_ref[...], preferred_element_type=jnp.float32) + b2_ref[...]
    if ln:
        h = _ln(h, g_ref[...], beta_ref[...])
    o_ref[...] = h


def _mlp(x, w1, b1, w2, b2, g, beta, ln):
    full = lambda i: (0, 0)
    return pl.pallas_call(
        functools.partial(_mlp_body, ln=ln),
        grid=(N // BLK,),
        in_specs=[
            pl.BlockSpec((BLK, F), lambda i: (i, 0)),
            pl.BlockSpec((F, F), full),
            pl.BlockSpec((1, F), full),
            pl.BlockSpec((F, F), full),
            pl.BlockSpec((1, F), full),
            pl.BlockSpec((1, F), full),
            pl.BlockSpec((1, F), full),
        ],
        out_specs=pl.BlockSpec((BLK, F), lambda i: (i, 0)),
        out_shape=jax.ShapeDtypeStruct((N, F), jnp.float32),
    )(x, w1, b1.reshape(1, F), w2, b2.reshape(1, F), g.reshape(1, F), beta.reshape(1, F))


def _combine1_body(agg_ref, h_ref, wl_ref, wr_ref, b_ref, o_ref):
    v = jnp.dot(agg_ref[...], wl_ref[...], preferred_element_type=jnp.float32)
    v += jnp.dot(h_ref[...], wr_ref[...], preferred_element_type=jnp.float32)
    o_ref[...] = jnp.maximum(v + b_ref[...], 0.0)


def _combine1(agg_pad, h, wl, wr, b):
    full = lambda i: (0, 0)
    return pl.pallas_call(
        _combine1_body,
        grid=(N // BLK,),
        in_specs=[
            pl.BlockSpec((BLK, F), lambda i: (i, 0)),
            pl.BlockSpec((BLK, F), lambda i: (i, 0)),
            pl.BlockSpec((F, F), full),
            pl.BlockSpec((F, F), full),
            pl.BlockSpec((1, F), full),
        ],
        out_specs=pl.BlockSpec((BLK, F), lambda i: (i, 0)),
        out_shape=jax.ShapeDtypeStruct((N, F), jnp.float32),
    )(agg_pad, h, wl, wr, b.reshape(1, F))


def _combine2_body(agg_ref, h1_ref, res_ref, wl_ref, wr_ref, b_ref, g_ref, beta_ref, o_ref):
    v = jnp.dot(agg_ref[...], wl_ref[...], preferred_element_type=jnp.float32)
    v += jnp.dot(h1_ref[...], wr_ref[...], preferred_element_type=jnp.float32)
    v += b_ref[...] + res_ref[...]
    o_ref[...] = _ln(v, g_ref[...], beta_ref[...])


def _combine2(agg_pad, h1, res, wl, wr, b, g, beta):
    full = lambda i: (0, 0)
    return pl.pallas_call(
        _combine2_body,
        grid=(N // BLK,),
        in_specs=[
            pl.BlockSpec((BLK, F), lambda i: (i, 0)),
            pl.BlockSpec((BLK, F), lambda i: (i, 0)),
            pl.BlockSpec((BLK, F), lambda i: (i, 0)),
            pl.BlockSpec((F, F), full),
            pl.BlockSpec((F, F), full),
            pl.BlockSpec((1, F), full),
            pl.BlockSpec((1, F), full),
            pl.BlockSpec((1, F), full),
        ],
        out_specs=pl.BlockSpec((BLK, F), lambda i: (i, 0)),
        out_shape=jax.ShapeDtypeStruct((N, F), jnp.float32),
    )(agg_pad, h1, res, wl, wr, b.reshape(1, F), g.reshape(1, F), beta.reshape(1, F))


def kernel(x, edge_index, enc_W1, enc_b1, enc_W2, enc_b2, enc_g, enc_beta,
           proc_Wl1, proc_bl1, proc_Wr1, proc_Wl2, proc_bl2, proc_Wr2,
           proc_g, proc_beta, dec_W1, dec_b1, dec_W2, dec_b2):
    src = edge_index[0]
    dst = edge_index[1]

    dst_s, src_s = lax.sort((dst, src), num_keys=1)
    first = jnp.concatenate([
        jnp.ones((1,), jnp.int32),
        (dst_s[1:] != dst_s[:-1]).astype(jnp.int32),
    ])
    last = jnp.concatenate([
        (dst_s[1:] != dst_s[:-1]).astype(jnp.int32),
        jnp.ones((1,), jnp.int32),
    ])
    row = jnp.mod(dst_s, ROWS)
    meta = (row << 2) | (last << 1) | first
    pad = jnp.zeros((CAP,), jnp.int32)
    src_pad = jnp.concatenate([src_s, pad])
    meta_pad = jnp.concatenate([meta, pad])

    bounds = (ROWS * jnp.arange(33, dtype=jnp.int32)).astype(dst_s.dtype)
    edges_at = jnp.searchsorted(dst_s, bounds, side="left").astype(jnp.int32)
    starts, ends = edges_at[:32], edges_at[1:33]
    se_arr = jnp.concatenate(
        [jnp.repeat(starts[:, None], 16, 1), jnp.repeat(ends[:, None], 16, 1)],
        axis=1,
    ).reshape(-1)

    ones = jnp.ones((F,), jnp.float32)
    zeros = jnp.zeros((F,), jnp.float32)

    h = _mlp(x, enc_W1, enc_b1, enc_W2, enc_b2, enc_g, enc_beta, ln=True)
    for i in range(3):
        agg1 = _seg_max(h, src_pad, meta_pad, se_arr)
        h1 = _combine1(agg1, h, proc_Wl1[i], proc_Wr1[i], proc_bl1[i])
        agg2 = _seg_max(h1, src_pad, meta_pad, se_arr)
        h = _combine2(agg2, h1, h, proc_Wl2[i], proc_Wr2[i], proc_bl2[i],
                      proc_g[i], proc_beta[i])
    return _mlp(h, dec_W1, dec_b1, dec_W2, dec_b2, ones, zeros, ln=False)


# speedup vs baseline: 8.8978x; 1.0200x over previous
import functools

import jax
import jax.numpy as jnp
from jax import lax
from jax.experimental import pallas as pl
from jax.experimental.pallas import tpu as pltpu
from jax.experimental.pallas import tpu_sc as plsc

N = 10000
E = 320000
F = 128
NW = 32
ROWS = 320
NPAD = NW * ROWS
CAP = 16384
CH = 128
BLK = 2000


def _build_seg_max():
    mesh = plsc.VectorSubcoreMesh(core_axis_name="c", subcore_axis_name="s")

    @functools.partial(
        pl.kernel,
        mesh=mesh,
        compiler_params=pltpu.CompilerParams(needs_layout_passes=False),
        out_type=jax.ShapeDtypeStruct((NPAD, F), jnp.float32),
        scratch_types=[
            pltpu.VMEM((ROWS, F), jnp.float32),
            pltpu.VMEM((CH, F), jnp.float32),
            pltpu.VMEM((CH, F), jnp.float32),
            pltpu.VMEM((CAP,), jnp.int32),
            pltpu.VMEM((CAP,), jnp.int32),
            pltpu.VMEM((32,), jnp.int32),
            pltpu.SemaphoreType.DMA,
            pltpu.SemaphoreType.DMA,
        ],
    )
    def seg_max(h_hbm, src_hbm, meta_hbm, se_hbm, out_hbm,
                agg, msgA, msgB, srcs, meta, se, semA, semB):
        cid = lax.axis_index("c")
        sid = lax.axis_index("s")
        t = sid * 2 + cid

        pltpu.sync_copy(se_hbm.at[pl.ds(pl.multiple_of(t * 32, 32), 32)], se)
        sev0 = se[0:16]
        sev1 = se[16:32]
        start = sev0[0]
        end = sev1[0]
        start8 = pl.multiple_of(start & jnp.int32(-8), 8)
        nsc = (end - start8 + (CAP - 1)) >> 14

        zv = jnp.zeros((16,), jnp.float32)

        def zrow(i, carry):
            for f in range(8):
                agg[i, pl.ds(f * 16, 16)] = zv
            return carry

        lax.fori_loop(0, ROWS, zrow, 0)

        negv = jnp.full((16,), -jnp.inf, jnp.float32)
        iota = lax.iota(jnp.int32, 16)
        startv = jnp.full((16,), start, jnp.int32)
        endv = jnp.full((16,), end, jnp.int32)

        def fire(c, msgbuf, sem):
            pltpu.async_copy(h_hbm.at[srcs.at[pl.ds(c * CH, CH)]], msgbuf, sem)

        def wait(c, msgbuf, sem):
            pltpu.make_async_copy(h_hbm.at[srcs.at[pl.ds(c * CH, CH)]], msgbuf, sem).wait()

        def sc_body(s, acc):
            soff = pl.multiple_of(start8 + s * CAP, 8)
            pltpu.sync_copy(src_hbm.at[pl.ds(soff, CAP)], srcs)
            pltpu.sync_copy(meta_hbm.at[pl.ds(soff, CAP)], meta)
            nch = jnp.minimum((end - soff + (CH - 1)) >> 7, CAP // CH)

            @pl.when(nch > 0)
            def _():
                fire(0, msgA, semA)

            def compute(c, msgbuf, acc):
                base = c * CH

                def body(jl, acc):
                    le = base + jl
                    jv = jnp.full((16,), le, jnp.int32)
                    m = plsc.load_gather(meta, [jv])
                    ev = jnp.full((16,), soff + le, jnp.int32)
                    validm = (ev >= startv) & (ev < endv)
                    firstm = (m & 1) > 0
                    lastm = (m & 2) > 0
                    vmask = lastm & validm
                    row = m >> 2
                    nacc = []
                    for f in range(8):
                        v = msgbuf[jl, pl.ds(f * 16, 16)]
                        a = jnp.maximum(jnp.where(firstm, negv, acc[f]), v)
                        plsc.store_scatter(agg, [row, iota + f * 16], a, mask=vmask)
                        nacc.append(a)
                    return tuple(nacc)

                return plsc.parallel_loop(0, CH, 1, unroll=4, carry=acc)(body)

            def pair(c2, acc):
                c0 = c2 * 2
                c1 = c0 + 1

                @pl.when(c1 < nch)
                def _():
                    fire(c1, msgB, semB)

                wait(c0, msgA, semA)
                acc = compute(c0, msgA, acc)

                @pl.when(c0 + 2 < nch)
                def _():
                    fire(c0 + 2, msgA, semA)

                def doB(a):
                    wait(c1, msgB, semB)
                    return compute(c1, msgB, a)

                return lax.cond(c1 < nch, doB, lambda a: a, acc)

            return lax.fori_loop(0, (nch + 1) >> 1, pair, acc)

        lax.fori_loop(0, nsc, sc_body, (negv,) * 8)
        pltpu.sync_copy(agg, out_hbm.at[pl.ds(pl.multiple_of(t * ROWS, ROWS), ROWS)])

    return seg_max


_seg_max = _build_seg_max()


def _ln(h, g, b):
    mu = jnp.mean(h, axis=-1, keepdims=True)
    d = h - mu
    var = jnp.mean(d * d, axis=-1, keepdims=True)
    return d * lax.rsqrt(var + 1e-5) * g + b


def _mlp_body(x_ref, w1_ref, b1_ref, w2_ref, b2_ref, g_ref, beta_ref, o_ref, *, ln):
    h = jnp.dot(x_ref[...], w1_ref[...], preferred_element_type=jnp.float32)
    h = jnp.maximum(h + b1_ref[...], 0.0)
    h = jnp.dot(h, w2_ref[...], preferred_element_type=jnp.float32) + b2_ref[...]
    if ln:
        h = _ln(h, g_ref[...], beta_ref[...])
    o_ref[...] = h


def _mlp(x, w1, b1, w2, b2, g, beta, ln):
    full = lambda i: (0, 0)
    return pl.pallas_call(
        functools.partial(_mlp_body, ln=ln),
        grid=(N // BLK,),
        in_specs=[
            pl.BlockSpec((BLK, F), lambda i: (i, 0)),
            pl.BlockSpec((F, F), full),
            pl.BlockSpec((1, F), full),
            pl.BlockSpec((F, F), full),
            pl.BlockSpec((1, F), full),
            pl.BlockSpec((1, F), full),
            pl.BlockSpec((1, F), full),
        ],
        out_specs=pl.BlockSpec((BLK, F), lambda i: (i, 0)),
        out_shape=jax.ShapeDtypeStruct((N, F), jnp.float32),
    )(x, w1, b1.reshape(1, F), w2, b2.reshape(1, F), g.reshape(1, F), beta.reshape(1, F))


def _combine1_body(agg_ref, h_ref, wl_ref, wr_ref, b_ref, o_ref):
    v = jnp.dot(agg_ref[...], wl_ref[...], preferred_element_type=jnp.float32)
    v += jnp.dot(h_ref[...], wr_ref[...], preferred_element_type=jnp.float32)
    o_ref[...] = jnp.maximum(v + b_ref[...], 0.0)


def _combine1(agg_pad, h, wl, wr, b):
    full = lambda i: (0, 0)
    return pl.pallas_call(
        _combine1_body,
        grid=(N // BLK,),
        in_specs=[
            pl.BlockSpec((BLK, F), lambda i: (i, 0)),
            pl.BlockSpec((BLK, F), lambda i: (i, 0)),
            pl.BlockSpec((F, F), full),
            pl.BlockSpec((F, F), full),
            pl.BlockSpec((1, F), full),
        ],
        out_specs=pl.BlockSpec((BLK, F), lambda i: (i, 0)),
        out_shape=jax.ShapeDtypeStruct((N, F), jnp.float32),
    )(agg_pad, h, wl, wr, b.reshape(1, F))


def _combine2_body(agg_ref, h1_ref, res_ref, wl_ref, wr_ref, b_ref, g_ref, beta_ref, o_ref):
    v = jnp.dot(agg_ref[...], wl_ref[...], preferred_element_type=jnp.float32)
    v += jnp.dot(h1_ref[...], wr_ref[...], preferred_element_type=jnp.float32)
    v += b_ref[...] + res_ref[...]
    o_ref[...] = _ln(v, g_ref[...], beta_ref[...])


def _combine2(agg_pad, h1, res, wl, wr, b, g, beta):
    full = lambda i: (0, 0)
    return pl.pallas_call(
        _combine2_body,
        grid=(N // BLK,),
        in_specs=[
            pl.BlockSpec((BLK, F), lambda i: (i, 0)),
            pl.BlockSpec((BLK, F), lambda i: (i, 0)),
            pl.BlockSpec((BLK, F), lambda i: (i, 0)),
            pl.BlockSpec((F, F), full),
            pl.BlockSpec((F, F), full),
            pl.BlockSpec((1, F), full),
            pl.BlockSpec((1, F), full),
            pl.BlockSpec((1, F), full),
        ],
        out_specs=pl.BlockSpec((BLK, F), lambda i: (i, 0)),
        out_shape=jax.ShapeDtypeStruct((N, F), jnp.float32),
    )(agg_pad, h1, res, wl, wr, b.reshape(1, F), g.reshape(1, F), beta.reshape(1, F))


def kernel(x, edge_index, enc_W1, enc_b1, enc_W2, enc_b2, enc_g, enc_beta,
           proc_Wl1, proc_bl1, proc_Wr1, proc_Wl2, proc_bl2, proc_Wr2,
           proc_g, proc_beta, dec_W1, dec_b1, dec_W2, dec_b2):
    src = edge_index[0]
    dst = edge_index[1]

    key = lax.sort((dst << 14) | src)
    dst_s = key >> 14
    src_s = key & jnp.int32(0x3FFF)
    first = jnp.concatenate([
        jnp.ones((1,), jnp.int32),
        (dst_s[1:] != dst_s[:-1]).astype(jnp.int32),
    ])
    last = jnp.concatenate([
        (dst_s[1:] != dst_s[:-1]).astype(jnp.int32),
        jnp.ones((1,), jnp.int32),
    ])
    row = jnp.mod(dst_s, ROWS)
    meta = (row << 2) | (last << 1) | first
    pad = jnp.zeros((CAP,), jnp.int32)
    src_pad = jnp.concatenate([src_s, pad])
    meta_pad = jnp.concatenate([meta, pad])

    bounds = (ROWS * jnp.arange(33, dtype=jnp.int32)).astype(dst_s.dtype)
    edges_at = jnp.searchsorted(dst_s, bounds, side="left").astype(jnp.int32)
    starts, ends = edges_at[:32], edges_at[1:33]
    se_arr = jnp.concatenate(
        [jnp.repeat(starts[:, None], 16, 1), jnp.repeat(ends[:, None], 16, 1)],
        axis=1,
    ).reshape(-1)

    ones = jnp.ones((F,), jnp.float32)
    zeros = jnp.zeros((F,), jnp.float32)

    h = _mlp(x, enc_W1, enc_b1, enc_W2, enc_b2, enc_g, enc_beta, ln=True)
    for i in range(3):
        agg1 = _seg_max(h, src_pad, meta_pad, se_arr)
        h1 = _combine1(agg1, h, proc_Wl1[i], proc_Wr1[i], proc_bl1[i])
        agg2 = _seg_max(h1, src_pad, meta_pad, se_arr)
        h = _combine2(agg2, h1, h, proc_Wl2[i], proc_Wr2[i], proc_bl2[i],
                      proc_g[i], proc_beta[i])
    return _mlp(h, dec_W1, dec_b1, dec_W2, dec_b2, ones, zeros, ln=False)
